# baseline (speedup 1.0000x reference)
import jax
import jax.numpy as jnp
from jax import lax
from jax.experimental import pallas as pl
from jax.experimental.pallas import tpu as pltpu
from jax.experimental.pallas import tpu_sc as plsc

B, N, C = 64, 576, 768
NC_TOT = N * C
K_TOTAL = int(B * N * C * 0.1)
KP = -(-K_TOTAL // B)
NSC = 16
CH = NC_TOT // NSC
NV = CH // 16
HB = 2048
LEVELS = ((20, 2048), (10, 1024), (0, 1024))


def _score_kernel(w_e_ref, w_o_ref, mt_ref, b_e_ref, b_o_ref, u_ref):
    t0 = jnp.dot(w_e_ref[...], mt_ref[...], preferred_element_type=jnp.float32) + b_e_ref[...]
    t1 = jnp.dot(w_o_ref[...], mt_ref[...], preferred_element_type=jnp.float32) + b_o_ref[...]
    mx = jnp.maximum(t0, t1)
    s0 = t0 - mx
    s1 = t1 - mx
    lse = jnp.log(jnp.exp(s0) + jnp.exp(s1))
    lp0 = s0 - lse
    lp1 = s1 - lse
    mx2 = jnp.maximum(lp0, lp1)
    e0 = jnp.exp(lp0 - mx2)
    e1 = jnp.exp(lp1 - mx2)
    s = e0 / (e0 + e1)
    u_ref[...] = jax.lax.bitcast_convert_type(s, jnp.int32)


def _sc_rank_kernel(u_hbm, nk_hbm, data_v, hist_v, hist_m, hist_all, outb_v,
                    tvec_v, shist_s, stied_s):
    wid = lax.axis_index("s")
    cid = lax.axis_index("c")
    base = wid * CH
    pltpu.sync_copy(u_hbm.at[pl.ds(base, CH)], data_v)

    iota16 = lax.iota(jnp.int32, 16)
    ones16 = jnp.ones((16,), jnp.int32)
    zeros16 = jnp.zeros((16,), jnp.int32)

    def zero_buf(ref, nwords):
        def zb(i, carry):
            ref[pl.ds(i * 16, 16)] = zeros16
            return carry
        lax.fori_loop(0, nwords // 16, zb, 0)

    vpref = jnp.int32(0)
    g_run = jnp.int32(0)
    m_cnt = jnp.int32(0)

    for lvl, (shift, nbins) in enumerate(LEVELS):
        bmask = jnp.int32(nbins - 1)
        himask = jnp.int32(-(1 << (shift + (11 if lvl == 0 else 10))) & 0x7FFFFFFF)

        zero_buf(hist_v, HB)

        def scan_body(i, carry, _shift=shift, _bmask=bmask, _himask=himask,
                      _vpref=vpref):
            vec = data_v[pl.ds(i * 16, 16)]
            inr = (vec & _himask) == _vpref
            idx = lax.shift_right_logical(vec, _shift) & _bmask
            idx = jnp.where(inr, idx, jnp.int32(HB - 1))
            plsc.addupdate_scatter(hist_v, [idx], ones16)
            return carry
        lax.fori_loop(0, NV, scan_body, 0)

        pltpu.sync_copy(hist_v, shist_s.at[wid])
        plsc.subcore_barrier()
        pltpu.sync_copy(shist_s, hist_all)

        def merge_body(i, carry):
            acc = hist_all[0, pl.ds(i * 16, 16)]
            for w in range(1, NSC):
                acc = acc + hist_all[w, pl.ds(i * 16, 16)]
            hist_m[pl.ds(i * 16, 16)] = acc
            return carry
        lax.fori_loop(0, nbins // 16, merge_body, 0)
        plsc.subcore_barrier()

        nvb = nbins // 16

        def bin_body(k, carry):
            grun, found, bstar, hstar = carry
            i = nvb - 1 - k
            vec = hist_m[pl.ds(i * 16, 16)]
            rev = lax.rev(vec, (0,))
            cs = plsc.cumsum(rev)
            cond = (grun + cs) >= KP
            pc = jnp.max(plsc.all_reduce_population_count(cond))
            kf = 16 - pc
            sel = iota16 == kf
            csk = jnp.sum(jnp.where(sel, cs, 0))
            hv = jnp.sum(jnp.where(sel, rev, 0))
            tot = jnp.sum(vec)
            this_found = (pc > 0).astype(jnp.int32)
            newly = this_found * (1 - found)
            bidx = i * 16 + 15 - kf
            bstar = jnp.where(newly == 1, bidx, bstar)
            hstar = jnp.where(newly == 1, hv, hstar)
            grun = jnp.where(newly == 1, grun + csk - hv,
                             jnp.where(found == 1, grun, grun + tot))
            found = jnp.maximum(found, this_found)
            return (grun, found, bstar, hstar)

        g_run, _found, bstar, hstar = lax.fori_loop(
            0, nvb, bin_body, (g_run, jnp.int32(0), jnp.int32(0), jnp.int32(0)))
        vpref = vpref | lax.shift_left(bstar, jnp.int32(shift))
        m_cnt = hstar

    v = vpref
    g = g_run
    m = m_cnt
    r = jnp.int32(K_TOTAL) - 64 * g

    def tie_body(i, tl):
        vec = data_v[pl.ds(i * 16, 16)]
        pc = jnp.max(plsc.all_reduce_population_count(vec == v))
        return tl + pc
    tl = lax.fori_loop(0, NV, tie_body, jnp.int32(0))

    tvec_v[...] = jnp.where(iota16 == 0, tl, 0)
    pltpu.sync_copy(tvec_v, stied_s.at[wid])
    plsc.subcore_barrier()

    prev = jnp.int32(0)
    for w in range(NSC):
        pltpu.sync_copy(stied_s.at[w], tvec_v)
        lane0 = jnp.sum(jnp.where(iota16 == 0, tvec_v[...], 0))
        prev = prev + jnp.where(jnp.int32(w) < wid, lane0, 0)

    def out_body(i, jrun):
        vec = data_v[pl.ds(i * 16, 16)]
        gt = vec > v
        tm = vec == v
        tmi = tm.astype(jnp.int32)
        csum = plsc.cumsum(tmi)
        jvec = prev + jrun + (csum - tmi)
        q = jnp.clip((r - jvec + m - 1) // m, 0, 64)
        n = jnp.where(gt, 64, jnp.where(tm, q, 0))
        outb_v[pl.ds(i * 16, 16)] = n
        pc = jnp.max(plsc.all_reduce_population_count(tm))
        return jrun + pc
    lax.fori_loop(0, NV, out_body, jnp.int32(0))

    @pl.when(cid == 0)
    def _write():
        pltpu.sync_copy(outb_v, nk_hbm.at[pl.ds(base, CH)])


def _mask_ln_kernel(x_ref, tok_ref, nk_ref, g_ref, b_ref, out_ref, mask_ref):
    b = pl.program_id(0)
    keep = nk_ref[...] > b
    keep3 = keep[None, :, :]
    xm = jnp.where(keep3, x_ref[...], tok_ref[...])
    mask_ref[...] = keep3.astype(jnp.float32)
    mu = jnp.mean(xm, axis=-1, keepdims=True)
    d = xm - mu
    var = jnp.mean(d * d, axis=-1, keepdims=True)
    out_ref[...] = d / jnp.sqrt(var + 1e-5) * g_ref[...] + b_ref[...]


def kernel(x, patch_mask_para, fc_W, fc_b, learnable_token, ln_gamma, ln_beta):
    mt = patch_mask_para.transpose(0, 2, 1).reshape(2 * N, C)
    w_e = fc_W[0::2, :]
    w_o = fc_W[1::2, :]
    b_e = fc_b[0::2][:, None]
    b_o = fc_b[1::2][:, None]

    u = pl.pallas_call(
        _score_kernel,
        out_shape=jax.ShapeDtypeStruct((N, C), jnp.int32),
    )(w_e, w_o, mt, b_e, b_o)

    mesh = plsc.VectorSubcoreMesh(core_axis_name="c", subcore_axis_name="s")
    n_keep_flat = pl.kernel(
        _sc_rank_kernel,
        mesh=mesh,
        compiler_params=pltpu.CompilerParams(needs_layout_passes=False),
        out_type=jax.ShapeDtypeStruct((NC_TOT,), jnp.int32),
        scratch_types=[
            pltpu.VMEM((CH,), jnp.int32),
            pltpu.VMEM((HB,), jnp.int32),
            pltpu.VMEM((HB,), jnp.int32),
            pltpu.VMEM((NSC, HB), jnp.int32),
            pltpu.VMEM((CH,), jnp.int32),
            pltpu.VMEM((16,), jnp.int32),
            pltpu.VMEM_SHARED((NSC, HB), jnp.int32),
            pltpu.VMEM_SHARED((NSC, 16), jnp.int32),
        ],
    )(u.reshape(NC_TOT))
    n_keep = n_keep_flat.reshape(N, C)

    out, mask = pl.pallas_call(
        _mask_ln_kernel,
        grid=(B,),
        in_specs=[
            pl.BlockSpec((1, N, C), lambda b: (b, 0, 0)),
            pl.BlockSpec((1, N, C), lambda b: (0, 0, 0)),
            pl.BlockSpec((N, C), lambda b: (0, 0)),
            pl.BlockSpec((1, C), lambda b: (0, 0)),
            pl.BlockSpec((1, C), lambda b: (0, 0)),
        ],
        out_specs=[
            pl.BlockSpec((1, N, C), lambda b: (b, 0, 0)),
            pl.BlockSpec((1, N, C), lambda b: (b, 0, 0)),
        ],
        out_shape=[
            jax.ShapeDtypeStruct((B, N, C), jnp.float32),
            jax.ShapeDtypeStruct((B, N, C), jnp.float32),
        ],
    )(x, learnable_token, n_keep, ln_gamma[None, :], ln_beta[None, :])
    return out, mask


# speedup vs baseline: 1.5211x; 1.5211x over previous
import jax
import jax.numpy as jnp
from jax import lax
from jax.experimental import pallas as pl
from jax.experimental.pallas import tpu as pltpu
from jax.experimental.pallas import tpu_sc as plsc

B, N, C = 64, 576, 768
NC_TOT = N * C
K_TOTAL = int(B * N * C * 0.1)
KP = -(-K_TOTAL // B)
NSC = 16
CH = NC_TOT // NSC
NV = CH // 16
HB = 2048
LEVELS = ((20, 2048), (10, 1024), (0, 1024))


def _score_kernel(w_e_ref, w_o_ref, mt_ref, b_e_ref, b_o_ref, u_ref):
    t0 = jnp.dot(w_e_ref[...], mt_ref[...], preferred_element_type=jnp.float32) + b_e_ref[...]
    t1 = jnp.dot(w_o_ref[...], mt_ref[...], preferred_element_type=jnp.float32) + b_o_ref[...]
    mx = jnp.maximum(t0, t1)
    s0 = t0 - mx
    s1 = t1 - mx
    lse = jnp.log(jnp.exp(s0) + jnp.exp(s1))
    lp0 = s0 - lse
    lp1 = s1 - lse
    mx2 = jnp.maximum(lp0, lp1)
    e0 = jnp.exp(lp0 - mx2)
    e1 = jnp.exp(lp1 - mx2)
    s = e0 / (e0 + e1)
    u_ref[...] = jax.lax.bitcast_convert_type(s, jnp.int32)


def _sc_rank_kernel(u_hbm, nk_hbm, data_v, hist_v, hist_m, hist_all, outb_v,
                    tvec_v, tied_all, shist_s, stied_s):
    wid = lax.axis_index("s")
    cid = lax.axis_index("c")
    base = wid * CH
    pltpu.sync_copy(u_hbm.at[pl.ds(base, CH)], data_v)

    iota16 = lax.iota(jnp.int32, 16)
    ones16 = jnp.ones((16,), jnp.int32)
    zeros16 = jnp.zeros((16,), jnp.int32)

    def zero_buf(ref, nwords):
        @plsc.parallel_loop(0, nwords // 16, unroll=8)
        def _zb(i):
            ref[pl.ds(i * 16, 16)] = zeros16

    vpref = jnp.int32(0)
    g_run = jnp.int32(0)
    m_cnt = jnp.int32(0)

    for lvl, (shift, nbins) in enumerate(LEVELS):
        bmask = jnp.int32(nbins - 1)
        himask = jnp.int32(-(1 << (shift + (11 if lvl == 0 else 10))) & 0x7FFFFFFF)

        zero_buf(hist_v, HB)

        @plsc.parallel_loop(0, NV, unroll=8)
        def scan_body(i, _shift=shift, _bmask=bmask, _himask=himask,
                      _vpref=vpref):
            vec = data_v[pl.ds(i * 16, 16)]
            inr = (vec & _himask) == _vpref
            idx = lax.shift_right_logical(vec, _shift) & _bmask
            idx = jnp.where(inr, idx, jnp.int32(HB - 1))
            plsc.addupdate_scatter(hist_v, [idx], ones16)

        pltpu.sync_copy(hist_v, shist_s.at[wid])
        plsc.subcore_barrier()
        pltpu.sync_copy(shist_s, hist_all)

        @plsc.parallel_loop(0, nbins // 16, unroll=2)
        def merge_body(i):
            acc = hist_all[0, pl.ds(i * 16, 16)]
            for w in range(1, NSC):
                acc = acc + hist_all[w, pl.ds(i * 16, 16)]
            hist_m[pl.ds(i * 16, 16)] = acc
        plsc.subcore_barrier()

        nvb = nbins // 16

        def bin_body(k, carry):
            grun, found, bstar, hstar = carry
            i = nvb - 1 - k
            vec = hist_m[pl.ds(i * 16, 16)]
            rev = lax.rev(vec, (0,))
            cs = plsc.cumsum(rev)
            cond = (grun + cs) >= KP
            pc = jnp.max(plsc.all_reduce_population_count(cond))
            kf = 16 - pc
            sel = iota16 == kf
            csk = jnp.sum(jnp.where(sel, cs, 0))
            hv = jnp.sum(jnp.where(sel, rev, 0))
            tot = jnp.sum(vec)
            this_found = (pc > 0).astype(jnp.int32)
            newly = this_found * (1 - found)
            bidx = i * 16 + 15 - kf
            bstar = jnp.where(newly == 1, bidx, bstar)
            hstar = jnp.where(newly == 1, hv, hstar)
            grun = jnp.where(newly == 1, grun + csk - hv,
                             jnp.where(found == 1, grun, grun + tot))
            found = jnp.maximum(found, this_found)
            return (grun, found, bstar, hstar)

        g_run, _found, bstar, hstar = lax.fori_loop(
            0, nvb, bin_body, (g_run, jnp.int32(0), jnp.int32(0), jnp.int32(0)))
        vpref = vpref | lax.shift_left(bstar, jnp.int32(shift))
        m_cnt = hstar

    v = vpref
    g = g_run
    m = m_cnt
    r = jnp.int32(K_TOTAL) - 64 * g

    tl_vec = plsc.load_gather(hist_v, [jnp.full((16,), 1, jnp.int32) * (v & jnp.int32(1023))])
    tl = jnp.max(tl_vec)

    tvec_v[...] = jnp.where(iota16 == 0, tl, 0)
    pltpu.sync_copy(tvec_v, stied_s.at[wid])
    plsc.subcore_barrier()
    pltpu.sync_copy(stied_s, tied_all)

    prev = jnp.int32(0)
    for w in range(NSC):
        lane0 = jnp.sum(jnp.where(iota16 == 0, tied_all[w, pl.ds(0, 16)], 0))
        prev = prev + jnp.where(jnp.int32(w) < wid, lane0, 0)

    a_q = lax.div(r, m)
    bmod = lax.rem(r, m)

    @plsc.parallel_loop(0, NV, unroll=8, carry=jnp.int32(0))
    def out_body(i, jrun):
        vec = data_v[pl.ds(i * 16, 16)]
        gt = vec > v
        tm = vec == v
        tmi = tm.astype(jnp.int32)
        csum = plsc.cumsum(tmi)
        jvec = prev + jrun + (csum - tmi)
        q = a_q + (jvec < bmod).astype(jnp.int32)
        n = jnp.where(gt, 64, jnp.where(tm, q, 0))
        outb_v[pl.ds(i * 16, 16)] = n
        pc = jnp.max(plsc.all_reduce_population_count(tm))
        return jrun + pc

    @pl.when(cid == 0)
    def _write():
        pltpu.sync_copy(outb_v, nk_hbm.at[pl.ds(base, CH)])


def _mask_ln_kernel(x_ref, tok_ref, nk_ref, g_ref, b_ref, out_ref, mask_ref):
    b = pl.program_id(0)
    keep = nk_ref[...] > b
    keep3 = keep[None, :, :]
    xm = jnp.where(keep3, x_ref[...], tok_ref[...])
    mask_ref[...] = keep3.astype(jnp.float32)
    mu = jnp.mean(xm, axis=-1, keepdims=True)
    d = xm - mu
    var = jnp.mean(d * d, axis=-1, keepdims=True)
    out_ref[...] = d / jnp.sqrt(var + 1e-5) * g_ref[...] + b_ref[...]


def kernel(x, patch_mask_para, fc_W, fc_b, learnable_token, ln_gamma, ln_beta):
    mt = patch_mask_para.transpose(0, 2, 1).reshape(2 * N, C)
    w_e = fc_W[0::2, :]
    w_o = fc_W[1::2, :]
    b_e = fc_b[0::2][:, None]
    b_o = fc_b[1::2][:, None]

    u = pl.pallas_call(
        _score_kernel,
        out_shape=jax.ShapeDtypeStruct((N, C), jnp.int32),
    )(w_e, w_o, mt, b_e, b_o)

    mesh = plsc.VectorSubcoreMesh(core_axis_name="c", subcore_axis_name="s")
    n_keep_flat = pl.kernel(
        _sc_rank_kernel,
        mesh=mesh,
        compiler_params=pltpu.CompilerParams(needs_layout_passes=False),
        out_type=jax.ShapeDtypeStruct((NC_TOT,), jnp.int32),
        scratch_types=[
            pltpu.VMEM((CH,), jnp.int32),
            pltpu.VMEM((HB,), jnp.int32),
            pltpu.VMEM((HB,), jnp.int32),
            pltpu.VMEM((NSC, HB), jnp.int32),
            pltpu.VMEM((CH,), jnp.int32),
            pltpu.VMEM((16,), jnp.int32),
            pltpu.VMEM((NSC, 16), jnp.int32),
            pltpu.VMEM_SHARED((NSC, HB), jnp.int32),
            pltpu.VMEM_SHARED((NSC, 16), jnp.int32),
        ],
    )(u.reshape(NC_TOT))
    n_keep = n_keep_flat.reshape(N, C)

    out, mask = pl.pallas_call(
        _mask_ln_kernel,
        grid=(B,),
        in_specs=[
            pl.BlockSpec((1, N, C), lambda b: (b, 0, 0)),
            pl.BlockSpec((1, N, C), lambda b: (0, 0, 0)),
            pl.BlockSpec((N, C), lambda b: (0, 0)),
            pl.BlockSpec((1, C), lambda b: (0, 0)),
            pl.BlockSpec((1, C), lambda b: (0, 0)),
        ],
        out_specs=[
            pl.BlockSpec((1, N, C), lambda b: (b, 0, 0)),
            pl.BlockSpec((1, N, C), lambda b: (b, 0, 0)),
        ],
        out_shape=[
            jax.ShapeDtypeStruct((B, N, C), jnp.float32),
            jax.ShapeDtypeStruct((B, N, C), jnp.float32),
        ],
    )(x, learnable_token, n_keep, ln_gamma[None, :], ln_beta[None, :])
    return out, mask
